# baseline (speedup 1.0000x reference)
import functools

import jax
import jax.numpy as jnp
from jax import lax
from jax.experimental import pallas as pl
from jax.experimental.pallas import tpu as pltpu
from jax.experimental.pallas import tpu_sc as plsc

INPUT_SIZE = 128
OUTPUT_SIZE = 100000
NUM_POS = 96
NUM_NOISE = 64
SFD = 8
N_CLASSES = 12
MOMENTUM = 0.5
B = 64

BM = 4096
N_NOISE_ROWS = NUM_NOISE * B
NOISE_END = NUM_POS + N_NOISE_ROWS


def _tc_body(xpos_ref, xn_ref, vis_ref, lab_ref, mem_ref,
             sim_ref, nsim_ref, lwo_ref, pos_ref):
    i = pl.program_id(0)
    f32 = jnp.float32

    xpos = xpos_ref[...]
    mem = mem_ref[...]

    sim_ref[...] = jax.lax.dot_general(
        xpos, mem, (((1,), (1,)), ((), ())), preferred_element_type=f32)

    @pl.when(i == 0)
    def _():
        lab = lab_ref[...]
        cls = jax.lax.broadcasted_iota(jnp.int32, (B, N_CLASSES), 1)
        eq = (lab == cls).astype(f32)
        cnt = jnp.sum(eq, axis=0, keepdims=True)
        denom = jnp.where(cnt == 0.0, 1.0, cnt)
        lwo = eq / denom
        lwo_ref[...] = lwo

        r0 = jax.lax.broadcasted_iota(jnp.int32, (NUM_POS, N_CLASSES), 0)
        r1 = jax.lax.broadcasted_iota(jnp.int32, (NUM_POS, N_CLASSES), 1)
        rrow = ((r0 // SFD) == r1).astype(f32)
        p1 = jax.lax.dot_general(rrow, lwo, (((1,), (1,)), ((), ())),
                                 preferred_element_type=f32)
        c0 = jax.lax.broadcasted_iota(jnp.int32, (B * SFD, B), 0)
        c1 = jax.lax.broadcasted_iota(jnp.int32, (B * SFD, B), 1)
        rcol = ((c0 // SFD) == c1).astype(f32)
        p2 = jax.lax.dot_general(p1, rcol, (((1,), (1,)), ((), ())),
                                 preferred_element_type=f32)
        m0 = jax.lax.broadcasted_iota(jnp.int32, (NUM_POS, B * SFD), 0)
        m1 = jax.lax.broadcasted_iota(jnp.int32, (NUM_POS, B * SFD), 1)
        pmat = p2 * ((m0 % SFD) == (m1 % SFD)).astype(f32)
        present = jnp.sum(pmat, axis=1, keepdims=True) > 0.5

        xv = xpos * vis_ref[...]
        get96 = jax.lax.dot_general(pmat, xv, (((1,), (0,)), ((), ())),
                                    preferred_element_type=f32)
        mem96 = mem[0:NUM_POS, :]
        pos_upd = MOMENTUM * mem96 + (1.0 - MOMENTUM) * jnp.where(
            present, get96, mem96)
        ss = jnp.sum(pos_upd * pos_upd, axis=1, keepdims=True)
        nrm = jnp.maximum(jnp.sqrt(ss), 1e-12)
        pos_ref[...] = pos_upd / nrm

        xn = xn_ref[...]
        nsim_ref[...] = jax.lax.dot_general(
            xn, mem96, (((1,), (1,)), ((), ())), preferred_element_type=f32)


_NC, _NS = 2, 16
_NW = _NC * _NS
XN_PER_W = N_NOISE_ROWS // _NW
MEM_ROWS = OUTPUT_SIZE - NOISE_END
CH = 256
N_CHUNKS = MEM_ROWS // CH
TAIL = MEM_ROWS - N_CHUNKS * CH


def _rsqrt16(x):
    i = plsc.bitcast(x, jnp.int32)
    i = jnp.int32(0x5F3759DF) - (i >> 1)
    y = plsc.bitcast(i, jnp.float32)
    for _ in range(3):
        y = y * (1.5 - 0.5 * x * y * y)
    return y


def _normalize_rows(buf, nrows):
    def row_body(r, carry):
        vecs = [buf[r, pl.ds(c * 16, 16)] for c in range(INPUT_SIZE // 16)]
        sq = [v * v for v in vecs]
        s0 = (sq[0] + sq[1]) + (sq[2] + sq[3])
        s1 = (sq[4] + sq[5]) + (sq[6] + sq[7])
        ss16 = s0 + s1
        ss = jnp.sum(ss16)
        ssv = jax.lax.broadcast_in_dim(ss, (16,), ())
        y = _rsqrt16(jnp.maximum(ssv, 1e-24))
        for c in range(INPUT_SIZE // 16):
            buf[r, pl.ds(c * 16, 16)] = vecs[c] * y
        return carry
    lax.fori_loop(0, nrows, row_body, jnp.int32(0))


def _sc_body(mem_hbm, xn_hbm, out_hbm, buf):
    wid = lax.axis_index("s") * _NC + lax.axis_index("c")

    xb = wid * XN_PER_W
    pltpu.sync_copy(xn_hbm.at[pl.ds(xb, XN_PER_W)], buf.at[pl.ds(0, XN_PER_W)])
    _normalize_rows(buf, XN_PER_W)
    pltpu.sync_copy(buf.at[pl.ds(0, XN_PER_W)],
                    out_hbm.at[pl.ds(NUM_POS + xb, XN_PER_W)])

    n_my = jnp.int32(N_CHUNKS // _NW) + (wid < (N_CHUNKS % _NW)).astype(jnp.int32)

    def chunk_body(t, carry):
        off = NOISE_END + (wid + t * _NW) * CH
        pltpu.sync_copy(mem_hbm.at[pl.ds(off, CH)], buf)
        _normalize_rows(buf, CH)
        pltpu.sync_copy(buf, out_hbm.at[pl.ds(off, CH)])
        return carry

    lax.fori_loop(0, n_my, chunk_body, jnp.int32(0))

    @pl.when(wid == _NW - 1)
    def _():
        off = NOISE_END + N_CHUNKS * CH
        pltpu.sync_copy(mem_hbm.at[pl.ds(off, TAIL)], buf.at[pl.ds(0, TAIL)])
        _normalize_rows(buf, TAIL)
        pltpu.sync_copy(buf.at[pl.ds(0, TAIL)], out_hbm.at[pl.ds(off, TAIL)])


_sc_update = functools.partial(
    pl.kernel,
    out_type=jax.ShapeDtypeStruct((OUTPUT_SIZE, INPUT_SIZE), jnp.float32),
    mesh=plsc.VectorSubcoreMesh(core_axis_name="c", subcore_axis_name="s",
                                num_cores=_NC, num_subcores=_NS),
    scratch_types=[pltpu.VMEM((CH, INPUT_SIZE), jnp.float32)],
    compiler_params=pltpu.CompilerParams(needs_layout_passes=False),
)(_sc_body)


def kernel(x, y, visible, img_label, memory):
    xpos = x[:, :SFD, :].reshape(B * SFD, INPUT_SIZE)
    xn = x[:, SFD:, :].reshape(B * NUM_NOISE, INPUT_SIZE)
    vis = visible.reshape(B * SFD, 1)
    lab = img_label.astype(jnp.int32).reshape(B, 1)

    grid = ((OUTPUT_SIZE + BM - 1) // BM,)
    sim, nsim, lwo, pos96 = pl.pallas_call(
        _tc_body,
        grid=grid,
        in_specs=[
            pl.BlockSpec((B * SFD, INPUT_SIZE), lambda i: (0, 0)),
            pl.BlockSpec((B * NUM_NOISE, INPUT_SIZE), lambda i: (0, 0)),
            pl.BlockSpec((B * SFD, 1), lambda i: (0, 0)),
            pl.BlockSpec((B, 1), lambda i: (0, 0)),
            pl.BlockSpec((BM, INPUT_SIZE), lambda i: (i, 0)),
        ],
        out_specs=[
            pl.BlockSpec((B * SFD, BM), lambda i: (0, i)),
            pl.BlockSpec((B * NUM_NOISE, NUM_POS), lambda i: (0, 0)),
            pl.BlockSpec((B, N_CLASSES), lambda i: (0, 0)),
            pl.BlockSpec((NUM_POS, INPUT_SIZE), lambda i: (0, 0)),
        ],
        out_shape=[
            jax.ShapeDtypeStruct((B * SFD, OUTPUT_SIZE), jnp.float32),
            jax.ShapeDtypeStruct((B * NUM_NOISE, NUM_POS), jnp.float32),
            jax.ShapeDtypeStruct((B, N_CLASSES), jnp.float32),
            jax.ShapeDtypeStruct((NUM_POS, INPUT_SIZE), jnp.float32),
        ],
    )(xpos, xn, vis, lab, memory)

    bank = _sc_update(memory, xn)
    new_memory = jax.lax.dynamic_update_slice(bank, pos96, (0, 0))

    similarity = sim.reshape(B, SFD, OUTPUT_SIZE)
    noise_similarity = nsim.reshape(B, NUM_NOISE, NUM_POS)
    y_idx = y.astype(jnp.int32)
    return (similarity, y_idx, noise_similarity, lwo, new_memory)


# speedup vs baseline: 1.3264x; 1.3264x over previous
import functools

import jax
import jax.numpy as jnp
from jax import lax
from jax.experimental import pallas as pl
from jax.experimental.pallas import tpu as pltpu
from jax.experimental.pallas import tpu_sc as plsc

INPUT_SIZE = 128
OUTPUT_SIZE = 100000
NUM_POS = 96
NUM_NOISE = 64
SFD = 8
N_CLASSES = 12
MOMENTUM = 0.5
B = 64

BM = 4096
N_NOISE_ROWS = NUM_NOISE * B
NOISE_END = NUM_POS + N_NOISE_ROWS


def _tc_body(xpos_ref, xn_ref, vis_ref, lab_ref, mem_ref,
             sim_ref, nsim_ref, lwo_ref, head_ref):
    i = pl.program_id(0)
    f32 = jnp.float32

    xpos = xpos_ref[...]
    mem = mem_ref[...]

    sim_ref[...] = jax.lax.dot_general(
        xpos, mem, (((1,), (1,)), ((), ())), preferred_element_type=f32)

    @pl.when(i == 0)
    def _():
        lab = lab_ref[...]
        cls = jax.lax.broadcasted_iota(jnp.int32, (B, N_CLASSES), 1)
        eq = (lab == cls).astype(f32)
        cnt = jnp.sum(eq, axis=0, keepdims=True)
        denom = jnp.where(cnt == 0.0, 1.0, cnt)
        lwo = eq / denom
        lwo_ref[...] = lwo

        r0 = jax.lax.broadcasted_iota(jnp.int32, (NUM_POS, N_CLASSES), 0)
        r1 = jax.lax.broadcasted_iota(jnp.int32, (NUM_POS, N_CLASSES), 1)
        rrow = ((r0 // SFD) == r1).astype(f32)
        p1 = jax.lax.dot_general(rrow, lwo, (((1,), (1,)), ((), ())),
                                 preferred_element_type=f32)
        c0 = jax.lax.broadcasted_iota(jnp.int32, (B * SFD, B), 0)
        c1 = jax.lax.broadcasted_iota(jnp.int32, (B * SFD, B), 1)
        rcol = ((c0 // SFD) == c1).astype(f32)
        p2 = jax.lax.dot_general(p1, rcol, (((1,), (1,)), ((), ())),
                                 preferred_element_type=f32)
        m0 = jax.lax.broadcasted_iota(jnp.int32, (NUM_POS, B * SFD), 0)
        m1 = jax.lax.broadcasted_iota(jnp.int32, (NUM_POS, B * SFD), 1)
        pmat = p2 * ((m0 % SFD) == (m1 % SFD)).astype(f32)
        present = jnp.sum(pmat, axis=1, keepdims=True) > 0.5

        xv = xpos * vis_ref[...]
        get96 = jax.lax.dot_general(pmat, xv, (((1,), (0,)), ((), ())),
                                    preferred_element_type=f32)
        mem96 = mem[0:NUM_POS, :]
        pos_upd = MOMENTUM * mem96 + (1.0 - MOMENTUM) * jnp.where(
            present, get96, mem96)

        xn = xn_ref[...]
        nsim_ref[...] = jax.lax.dot_general(
            xn, mem96, (((1,), (1,)), ((), ())), preferred_element_type=f32)

        upd = jnp.concatenate([pos_upd, xn], axis=0)
        ss = jnp.sum(upd * upd, axis=1, keepdims=True)
        nrm = jnp.maximum(jnp.sqrt(ss), 1e-12)
        head_ref[...] = upd / nrm


_NC, _NS = 2, 16
_NW = _NC * _NS
MEM_ROWS = OUTPUT_SIZE - NOISE_END
CH = 512
N_CHUNKS = MEM_ROWS // CH
TAIL = MEM_ROWS - N_CHUNKS * CH


def _normalize_rows(buf, nrows):
    @plsc.parallel_loop(0, nrows, 1, unroll=4)
    def _row(r):
        vecs = [buf[r, pl.ds(c * 16, 16)] for c in range(INPUT_SIZE // 16)]
        sq = [v * v for v in vecs]
        s0 = (sq[0] + sq[1]) + (sq[2] + sq[3])
        s1 = (sq[4] + sq[5]) + (sq[6] + sq[7])
        ss16 = s0 + s1
        ss = jnp.sum(ss16)
        ssv = jnp.maximum(jax.lax.broadcast_in_dim(ss, (16,), ()), 1e-24)
        i32 = plsc.bitcast(ssv, jnp.int32)
        y = plsc.bitcast(jnp.int32(0x5F3759DF) - (i32 >> 1), jnp.float32)
        y = y * (1.5 - 0.5 * ssv * y * y)
        y = y * (1.5 - 0.5 * ssv * y * y)
        for c in range(INPUT_SIZE // 16):
            buf[r, pl.ds(c * 16, 16)] = vecs[c] * y


def _sc_body(mem_hbm, out_hbm, buf):
    wid = lax.axis_index("s") * _NC + lax.axis_index("c")

    n_my = jnp.int32(N_CHUNKS // _NW) + (wid < (N_CHUNKS % _NW)).astype(jnp.int32)

    def chunk_body(t, carry):
        off = NOISE_END + (wid + t * _NW) * CH
        pltpu.sync_copy(mem_hbm.at[pl.ds(off, CH)], buf)
        _normalize_rows(buf, CH)
        pltpu.sync_copy(buf, out_hbm.at[pl.ds(off, CH)])
        return carry

    lax.fori_loop(0, n_my, chunk_body, jnp.int32(0))

    @pl.when(wid == _NW - 1)
    def _():
        off = NOISE_END + N_CHUNKS * CH
        pltpu.sync_copy(mem_hbm.at[pl.ds(off, TAIL)], buf.at[pl.ds(0, TAIL)])
        _normalize_rows(buf, TAIL)
        pltpu.sync_copy(buf.at[pl.ds(0, TAIL)], out_hbm.at[pl.ds(off, TAIL)])


_sc_update = functools.partial(
    pl.kernel,
    out_type=jax.ShapeDtypeStruct((OUTPUT_SIZE, INPUT_SIZE), jnp.float32),
    mesh=plsc.VectorSubcoreMesh(core_axis_name="c", subcore_axis_name="s",
                                num_cores=_NC, num_subcores=_NS),
    scratch_types=[pltpu.VMEM((CH, INPUT_SIZE), jnp.float32)],
    compiler_params=pltpu.CompilerParams(needs_layout_passes=False),
)(_sc_body)


def kernel(x, y, visible, img_label, memory):
    xpos = x[:, :SFD, :].reshape(B * SFD, INPUT_SIZE)
    xn = x[:, SFD:, :].reshape(B * NUM_NOISE, INPUT_SIZE)
    vis = visible.reshape(B * SFD, 1)
    lab = img_label.astype(jnp.int32).reshape(B, 1)

    grid = ((OUTPUT_SIZE + BM - 1) // BM,)
    sim, nsim, lwo, head = pl.pallas_call(
        _tc_body,
        grid=grid,
        in_specs=[
            pl.BlockSpec((B * SFD, INPUT_SIZE), lambda i: (0, 0)),
            pl.BlockSpec((B * NUM_NOISE, INPUT_SIZE), lambda i: (0, 0)),
            pl.BlockSpec((B * SFD, 1), lambda i: (0, 0)),
            pl.BlockSpec((B, 1), lambda i: (0, 0)),
            pl.BlockSpec((BM, INPUT_SIZE), lambda i: (i, 0)),
        ],
        out_specs=[
            pl.BlockSpec((B * SFD, BM), lambda i: (0, i)),
            pl.BlockSpec((B * NUM_NOISE, NUM_POS), lambda i: (0, 0)),
            pl.BlockSpec((B, N_CLASSES), lambda i: (0, 0)),
            pl.BlockSpec((NOISE_END, INPUT_SIZE), lambda i: (0, 0)),
        ],
        out_shape=[
            jax.ShapeDtypeStruct((B * SFD, OUTPUT_SIZE), jnp.float32),
            jax.ShapeDtypeStruct((B * NUM_NOISE, NUM_POS), jnp.float32),
            jax.ShapeDtypeStruct((B, N_CLASSES), jnp.float32),
            jax.ShapeDtypeStruct((NOISE_END, INPUT_SIZE), jnp.float32),
        ],
    )(xpos, xn, vis, lab, memory)

    bank = _sc_update(memory)
    new_memory = jax.lax.dynamic_update_slice(bank, head, (0, 0))

    similarity = sim.reshape(B, SFD, OUTPUT_SIZE)
    noise_similarity = nsim.reshape(B, NUM_NOISE, NUM_POS)
    y_idx = y.astype(jnp.int32)
    return (similarity, y_idx, noise_similarity, lwo, new_memory)


# speedup vs baseline: 1.3274x; 1.0008x over previous
import functools

import jax
import jax.numpy as jnp
from jax import lax
from jax.experimental import pallas as pl
from jax.experimental.pallas import tpu as pltpu
from jax.experimental.pallas import tpu_sc as plsc

INPUT_SIZE = 128
OUTPUT_SIZE = 100000
NUM_POS = 96
NUM_NOISE = 64
SFD = 8
N_CLASSES = 12
MOMENTUM = 0.5
B = 64

BM = 4096
N_NOISE_ROWS = NUM_NOISE * B
NOISE_END = NUM_POS + N_NOISE_ROWS


def _tc_body(xpos_ref, xn_ref, vis_ref, lab_ref, mem_ref,
             sim_ref, nsim_ref, lwo_ref, head_ref):
    i = pl.program_id(0)
    f32 = jnp.float32

    xpos = xpos_ref[...]
    mem = mem_ref[...]

    sim_ref[...] = jax.lax.dot_general(
        xpos.astype(jnp.bfloat16), mem.astype(jnp.bfloat16),
        (((1,), (1,)), ((), ())), preferred_element_type=f32)

    @pl.when(i == 0)
    def _():
        lab = lab_ref[...]
        cls = jax.lax.broadcasted_iota(jnp.int32, (B, N_CLASSES), 1)
        eq = (lab == cls).astype(f32)
        cnt = jnp.sum(eq, axis=0, keepdims=True)
        denom = jnp.where(cnt == 0.0, 1.0, cnt)
        lwo = eq / denom
        lwo_ref[...] = lwo

        r0 = jax.lax.broadcasted_iota(jnp.int32, (NUM_POS, N_CLASSES), 0)
        r1 = jax.lax.broadcasted_iota(jnp.int32, (NUM_POS, N_CLASSES), 1)
        rrow = ((r0 // SFD) == r1).astype(f32)
        p1 = jax.lax.dot_general(rrow, lwo, (((1,), (1,)), ((), ())),
                                 preferred_element_type=f32)
        c0 = jax.lax.broadcasted_iota(jnp.int32, (B * SFD, B), 0)
        c1 = jax.lax.broadcasted_iota(jnp.int32, (B * SFD, B), 1)
        rcol = ((c0 // SFD) == c1).astype(f32)
        p2 = jax.lax.dot_general(p1, rcol, (((1,), (1,)), ((), ())),
                                 preferred_element_type=f32)
        m0 = jax.lax.broadcasted_iota(jnp.int32, (NUM_POS, B * SFD), 0)
        m1 = jax.lax.broadcasted_iota(jnp.int32, (NUM_POS, B * SFD), 1)
        pmat = p2 * ((m0 % SFD) == (m1 % SFD)).astype(f32)
        present = jnp.sum(pmat, axis=1, keepdims=True) > 0.5

        xv = xpos * vis_ref[...]
        get96 = jax.lax.dot_general(pmat, xv, (((1,), (0,)), ((), ())),
                                    preferred_element_type=f32)
        mem96 = mem[0:NUM_POS, :]
        pos_upd = MOMENTUM * mem96 + (1.0 - MOMENTUM) * jnp.where(
            present, get96, mem96)

        xn = xn_ref[...]
        nsim_ref[...] = jax.lax.dot_general(
            xn, mem96, (((1,), (1,)), ((), ())), preferred_element_type=f32)

        upd = jnp.concatenate([pos_upd, xn], axis=0)
        ss = jnp.sum(upd * upd, axis=1, keepdims=True)
        nrm = jnp.maximum(jnp.sqrt(ss), 1e-12)
        head_ref[...] = upd / nrm


_NC, _NS = 2, 16
_NW = _NC * _NS
MEM_ROWS = OUTPUT_SIZE - NOISE_END
CH = 512
N_CHUNKS = MEM_ROWS // CH
TAIL = MEM_ROWS - N_CHUNKS * CH


def _normalize_rows(buf, nrows):
    @plsc.parallel_loop(0, nrows, 1, unroll=4)
    def _row(r):
        vecs = [buf[r, pl.ds(c * 16, 16)] for c in range(INPUT_SIZE // 16)]
        sq = [v * v for v in vecs]
        s0 = (sq[0] + sq[1]) + (sq[2] + sq[3])
        s1 = (sq[4] + sq[5]) + (sq[6] + sq[7])
        ss16 = s0 + s1
        ss = jnp.sum(ss16)
        ssv = jnp.maximum(jax.lax.broadcast_in_dim(ss, (16,), ()), 1e-24)
        i32 = plsc.bitcast(ssv, jnp.int32)
        y = plsc.bitcast(jnp.int32(0x5F3759DF) - (i32 >> 1), jnp.float32)
        y = y * (1.5 - 0.5 * ssv * y * y)
        y = y * (1.5 - 0.5 * ssv * y * y)
        for c in range(INPUT_SIZE // 16):
            buf[r, pl.ds(c * 16, 16)] = vecs[c] * y


def _sc_body(mem_hbm, out_hbm, buf):
    wid = lax.axis_index("s") * _NC + lax.axis_index("c")

    n_my = jnp.int32(N_CHUNKS // _NW) + (wid < (N_CHUNKS % _NW)).astype(jnp.int32)

    def chunk_body(t, carry):
        off = NOISE_END + (wid + t * _NW) * CH
        pltpu.sync_copy(mem_hbm.at[pl.ds(off, CH)], buf)
        _normalize_rows(buf, CH)
        pltpu.sync_copy(buf, out_hbm.at[pl.ds(off, CH)])
        return carry

    lax.fori_loop(0, n_my, chunk_body, jnp.int32(0))

    @pl.when(wid == _NW - 1)
    def _():
        off = NOISE_END + N_CHUNKS * CH
        pltpu.sync_copy(mem_hbm.at[pl.ds(off, TAIL)], buf.at[pl.ds(0, TAIL)])
        _normalize_rows(buf, TAIL)
        pltpu.sync_copy(buf.at[pl.ds(0, TAIL)], out_hbm.at[pl.ds(off, TAIL)])


_sc_update = functools.partial(
    pl.kernel,
    out_type=jax.ShapeDtypeStruct((OUTPUT_SIZE, INPUT_SIZE), jnp.float32),
    mesh=plsc.VectorSubcoreMesh(core_axis_name="c", subcore_axis_name="s",
                                num_cores=_NC, num_subcores=_NS),
    scratch_types=[pltpu.VMEM((CH, INPUT_SIZE), jnp.float32)],
    compiler_params=pltpu.CompilerParams(needs_layout_passes=False),
)(_sc_body)


def kernel(x, y, visible, img_label, memory):
    xpos = x[:, :SFD, :].reshape(B * SFD, INPUT_SIZE)
    xn = x[:, SFD:, :].reshape(B * NUM_NOISE, INPUT_SIZE)
    vis = visible.reshape(B * SFD, 1)
    lab = img_label.astype(jnp.int32).reshape(B, 1)

    grid = ((OUTPUT_SIZE + BM - 1) // BM,)
    sim, nsim, lwo, head = pl.pallas_call(
        _tc_body,
        grid=grid,
        in_specs=[
            pl.BlockSpec((B * SFD, INPUT_SIZE), lambda i: (0, 0)),
            pl.BlockSpec((B * NUM_NOISE, INPUT_SIZE), lambda i: (0, 0)),
            pl.BlockSpec((B * SFD, 1), lambda i: (0, 0)),
            pl.BlockSpec((B, 1), lambda i: (0, 0)),
            pl.BlockSpec((BM, INPUT_SIZE), lambda i: (i, 0)),
        ],
        out_specs=[
            pl.BlockSpec((B * SFD, BM), lambda i: (0, i)),
            pl.BlockSpec((B * NUM_NOISE, NUM_POS), lambda i: (0, 0)),
            pl.BlockSpec((B, N_CLASSES), lambda i: (0, 0)),
            pl.BlockSpec((NOISE_END, INPUT_SIZE), lambda i: (0, 0)),
        ],
        out_shape=[
            jax.ShapeDtypeStruct((B * SFD, OUTPUT_SIZE), jnp.float32),
            jax.ShapeDtypeStruct((B * NUM_NOISE, NUM_POS), jnp.float32),
            jax.ShapeDtypeStruct((B, N_CLASSES), jnp.float32),
            jax.ShapeDtypeStruct((NOISE_END, INPUT_SIZE), jnp.float32),
        ],
    )(xpos, xn, vis, lab, memory)

    bank = _sc_update(memory)
    new_memory = jax.lax.dynamic_update_slice(bank, head, (0, 0))

    similarity = sim.reshape(B, SFD, OUTPUT_SIZE)
    noise_similarity = nsim.reshape(B, NUM_NOISE, NUM_POS)
    y_idx = y.astype(jnp.int32)
    return (similarity, y_idx, noise_similarity, lwo, new_memory)


# speedup vs baseline: 1.3519x; 1.0184x over previous
import functools

import jax
import jax.numpy as jnp
from jax import lax
from jax.experimental import pallas as pl
from jax.experimental.pallas import tpu as pltpu
from jax.experimental.pallas import tpu_sc as plsc

INPUT_SIZE = 128
OUTPUT_SIZE = 100000
NUM_POS = 96
NUM_NOISE = 64
SFD = 8
N_CLASSES = 12
MOMENTUM = 0.5
B = 64

BM = 4096
N_NOISE_ROWS = NUM_NOISE * B
NOISE_END = NUM_POS + N_NOISE_ROWS


def _tc_body(xpos_ref, xn_ref, vis_ref, lab_ref, mem_ref,
             sim_ref, nsim_ref, lwo_ref, head_ref):
    i = pl.program_id(0)
    f32 = jnp.float32

    xpos = xpos_ref[...]
    mem = mem_ref[...]

    sim_ref[...] = jax.lax.dot_general(
        xpos.astype(jnp.bfloat16), mem.astype(jnp.bfloat16),
        (((1,), (1,)), ((), ())), preferred_element_type=f32)

    @pl.when(i == 0)
    def _():
        lab = lab_ref[...]
        cls = jax.lax.broadcasted_iota(jnp.int32, (B, N_CLASSES), 1)
        eq = (lab == cls).astype(f32)
        cnt = jnp.sum(eq, axis=0, keepdims=True)
        denom = jnp.where(cnt == 0.0, 1.0, cnt)
        lwo = eq / denom
        lwo_ref[...] = lwo

        r0 = jax.lax.broadcasted_iota(jnp.int32, (NUM_POS, N_CLASSES), 0)
        r1 = jax.lax.broadcasted_iota(jnp.int32, (NUM_POS, N_CLASSES), 1)
        rrow = ((r0 // SFD) == r1).astype(f32)
        p1 = jax.lax.dot_general(rrow, lwo, (((1,), (1,)), ((), ())),
                                 preferred_element_type=f32)
        c0 = jax.lax.broadcasted_iota(jnp.int32, (B * SFD, B), 0)
        c1 = jax.lax.broadcasted_iota(jnp.int32, (B * SFD, B), 1)
        rcol = ((c0 // SFD) == c1).astype(f32)
        p2 = jax.lax.dot_general(p1, rcol, (((1,), (1,)), ((), ())),
                                 preferred_element_type=f32)
        m0 = jax.lax.broadcasted_iota(jnp.int32, (NUM_POS, B * SFD), 0)
        m1 = jax.lax.broadcasted_iota(jnp.int32, (NUM_POS, B * SFD), 1)
        pmat = p2 * ((m0 % SFD) == (m1 % SFD)).astype(f32)
        present = jnp.sum(pmat, axis=1, keepdims=True) > 0.5

        xv = xpos * vis_ref[...]
        get96 = jax.lax.dot_general(pmat, xv, (((1,), (0,)), ((), ())),
                                    preferred_element_type=f32)
        mem96 = mem[0:NUM_POS, :]
        pos_upd = MOMENTUM * mem96 + (1.0 - MOMENTUM) * jnp.where(
            present, get96, mem96)

        xn = xn_ref[...]
        nsim_ref[...] = jax.lax.dot_general(
            xn, mem96, (((1,), (1,)), ((), ())), preferred_element_type=f32)

        upd = jnp.concatenate([pos_upd, xn], axis=0)
        ss = jnp.sum(upd * upd, axis=1, keepdims=True)
        nrm = jnp.maximum(jnp.sqrt(ss), 1e-12)
        head_ref[...] = upd / nrm


_NC, _NS = 2, 16
_NW = _NC * _NS
MEM_ROWS = OUTPUT_SIZE - NOISE_END
CH = 448
N_CHUNKS = MEM_ROWS // CH
TAIL = MEM_ROWS - N_CHUNKS * CH


def _normalize_rows(buf, base, nrows):
    @plsc.parallel_loop(0, nrows, 1, unroll=4)
    def _row(r):
        rr = base + r
        vecs = [buf[rr, pl.ds(c * 16, 16)] for c in range(INPUT_SIZE // 16)]
        sq = [v * v for v in vecs]
        s0 = (sq[0] + sq[1]) + (sq[2] + sq[3])
        s1 = (sq[4] + sq[5]) + (sq[6] + sq[7])
        ss16 = s0 + s1
        ss = jnp.sum(ss16)
        ssv = jnp.maximum(jax.lax.broadcast_in_dim(ss, (16,), ()), 1e-24)
        i32 = plsc.bitcast(ssv, jnp.int32)
        y = plsc.bitcast(jnp.int32(0x5F3759DF) - (i32 >> 1), jnp.float32)
        y = y * (1.5 - 0.5 * ssv * y * y)
        y = y * (1.5 - 0.5 * ssv * y * y)
        for c in range(INPUT_SIZE // 16):
            buf[rr, pl.ds(c * 16, 16)] = vecs[c] * y


def _sc_body(mem_hbm, out_hbm, buf, in_sems, out_sems):
    wid = lax.axis_index("s") * _NC + lax.axis_index("c")

    n_my = jnp.int32(N_CHUNKS // _NW) + (wid < (N_CHUNKS % _NW)).astype(jnp.int32)

    def off(t):
        return NOISE_END + (wid + t * _NW) * CH

    pltpu.async_copy(mem_hbm.at[pl.ds(off(0), CH)], buf.at[pl.ds(0, CH)],
                     in_sems.at[0])

    def chunk_body(t, carry):
        s = jnp.bitwise_and(t, 1)
        so = 1 - s
        pltpu.make_async_copy(mem_hbm.at[pl.ds(off(t), CH)],
                              buf.at[pl.ds(s * CH, CH)], in_sems.at[s]).wait()

        @pl.when(t >= 1)
        def _():
            pltpu.make_async_copy(buf.at[pl.ds(so * CH, CH)],
                                  out_hbm.at[pl.ds(off(t - 1), CH)],
                                  out_sems.at[so]).wait()

        @pl.when(t + 1 < n_my)
        def _():
            pltpu.async_copy(mem_hbm.at[pl.ds(off(t + 1), CH)],
                             buf.at[pl.ds(so * CH, CH)], in_sems.at[so])

        _normalize_rows(buf, s * CH, CH)
        pltpu.async_copy(buf.at[pl.ds(s * CH, CH)],
                         out_hbm.at[pl.ds(off(t), CH)], out_sems.at[s])
        return carry

    lax.fori_loop(0, n_my, chunk_body, jnp.int32(0))

    sl = jnp.bitwise_and(n_my - 1, 1)
    pltpu.make_async_copy(buf.at[pl.ds(sl * CH, CH)],
                          out_hbm.at[pl.ds(off(n_my - 1), CH)],
                          out_sems.at[sl]).wait()

    @pl.when(wid == _NW - 1)
    def _():
        toff = NOISE_END + N_CHUNKS * CH
        pltpu.sync_copy(mem_hbm.at[pl.ds(toff, TAIL)], buf.at[pl.ds(0, TAIL)])
        _normalize_rows(buf, 0, TAIL)
        pltpu.sync_copy(buf.at[pl.ds(0, TAIL)], out_hbm.at[pl.ds(toff, TAIL)])


_sc_update = functools.partial(
    pl.kernel,
    out_type=jax.ShapeDtypeStruct((OUTPUT_SIZE, INPUT_SIZE), jnp.float32),
    mesh=plsc.VectorSubcoreMesh(core_axis_name="c", subcore_axis_name="s",
                                num_cores=_NC, num_subcores=_NS),
    scratch_types=[pltpu.VMEM((2 * CH, INPUT_SIZE), jnp.float32),
                   pltpu.SemaphoreType.DMA((2,)),
                   pltpu.SemaphoreType.DMA((2,))],
    compiler_params=pltpu.CompilerParams(needs_layout_passes=False),
)(_sc_body)


def kernel(x, y, visible, img_label, memory):
    xpos = x[:, :SFD, :].reshape(B * SFD, INPUT_SIZE)
    xn = x[:, SFD:, :].reshape(B * NUM_NOISE, INPUT_SIZE)
    vis = visible.reshape(B * SFD, 1)
    lab = img_label.astype(jnp.int32).reshape(B, 1)

    grid = ((OUTPUT_SIZE + BM - 1) // BM,)
    sim, nsim, lwo, head = pl.pallas_call(
        _tc_body,
        grid=grid,
        in_specs=[
            pl.BlockSpec((B * SFD, INPUT_SIZE), lambda i: (0, 0)),
            pl.BlockSpec((B * NUM_NOISE, INPUT_SIZE), lambda i: (0, 0)),
            pl.BlockSpec((B * SFD, 1), lambda i: (0, 0)),
            pl.BlockSpec((B, 1), lambda i: (0, 0)),
            pl.BlockSpec((BM, INPUT_SIZE), lambda i: (i, 0)),
        ],
        out_specs=[
            pl.BlockSpec((B * SFD, BM), lambda i: (0, i)),
            pl.BlockSpec((B * NUM_NOISE, NUM_POS), lambda i: (0, 0)),
            pl.BlockSpec((B, N_CLASSES), lambda i: (0, 0)),
            pl.BlockSpec((NOISE_END, INPUT_SIZE), lambda i: (0, 0)),
        ],
        out_shape=[
            jax.ShapeDtypeStruct((B * SFD, OUTPUT_SIZE), jnp.float32),
            jax.ShapeDtypeStruct((B * NUM_NOISE, NUM_POS), jnp.float32),
            jax.ShapeDtypeStruct((B, N_CLASSES), jnp.float32),
            jax.ShapeDtypeStruct((NOISE_END, INPUT_SIZE), jnp.float32),
        ],
    )(xpos, xn, vis, lab, memory)

    bank = _sc_update(memory)
    new_memory = jax.lax.dynamic_update_slice(bank, head, (0, 0))

    similarity = sim.reshape(B, SFD, OUTPUT_SIZE)
    noise_similarity = nsim.reshape(B, NUM_NOISE, NUM_POS)
    y_idx = y.astype(jnp.int32)
    return (similarity, y_idx, noise_similarity, lwo, new_memory)


# speedup vs baseline: 1.3786x; 1.0197x over previous
import functools

import jax
import jax.numpy as jnp
from jax import lax
from jax.experimental import pallas as pl
from jax.experimental.pallas import tpu as pltpu
from jax.experimental.pallas import tpu_sc as plsc

INPUT_SIZE = 128
OUTPUT_SIZE = 100000
NUM_POS = 96
NUM_NOISE = 64
SFD = 8
N_CLASSES = 12
MOMENTUM = 0.5
B = 64

BM = 4096
N_NOISE_ROWS = NUM_NOISE * B
NOISE_END = NUM_POS + N_NOISE_ROWS


def _tc_body(x_ref, vis_ref, lab_ref, mem_ref,
             sim_ref, nsim_ref, lwo_ref, head_ref):
    i = pl.program_id(0)
    f32 = jnp.float32

    xpos = x_ref[:, 0:SFD, :]
    mem = mem_ref[...]

    sim_ref[...] = jax.lax.dot_general(
        xpos.astype(jnp.bfloat16), mem.astype(jnp.bfloat16),
        (((2,), (1,)), ((), ())), preferred_element_type=f32)

    @pl.when(i == 0)
    def _():
        lab = lab_ref[...]
        cls = jax.lax.broadcasted_iota(jnp.int32, (B, N_CLASSES), 1)
        eq = (lab == cls).astype(f32)
        cnt = jnp.sum(eq, axis=0, keepdims=True)
        denom = jnp.where(cnt == 0.0, 1.0, cnt)
        lwo = eq / denom
        lwo_ref[...] = lwo

        r0 = jax.lax.broadcasted_iota(jnp.int32, (NUM_POS, N_CLASSES), 0)
        r1 = jax.lax.broadcasted_iota(jnp.int32, (NUM_POS, N_CLASSES), 1)
        rrow = ((r0 // SFD) == r1).astype(f32)
        p1 = jax.lax.dot_general(rrow, lwo, (((1,), (1,)), ((), ())),
                                 preferred_element_type=f32)
        c0 = jax.lax.broadcasted_iota(jnp.int32, (B * SFD, B), 0)
        c1 = jax.lax.broadcasted_iota(jnp.int32, (B * SFD, B), 1)
        rcol = ((c0 // SFD) == c1).astype(f32)
        p2 = jax.lax.dot_general(p1, rcol, (((1,), (1,)), ((), ())),
                                 preferred_element_type=f32)
        m0 = jax.lax.broadcasted_iota(jnp.int32, (NUM_POS, B * SFD), 0)
        m1 = jax.lax.broadcasted_iota(jnp.int32, (NUM_POS, B * SFD), 1)
        pmat = p2 * ((m0 % SFD) == (m1 % SFD)).astype(f32)
        present = jnp.sum(pmat, axis=1, keepdims=True) > 0.5

        xv = (xpos * vis_ref[...][:, :, None]).reshape(B * SFD, INPUT_SIZE)
        get96 = jax.lax.dot_general(pmat, xv, (((1,), (0,)), ((), ())),
                                    preferred_element_type=f32)
        mem96 = mem[0:NUM_POS, :]
        pos_upd = MOMENTUM * mem96 + (1.0 - MOMENTUM) * jnp.where(
            present, get96, mem96)

        xn = x_ref[:, SFD:, :]
        nsim_ref[...] = jax.lax.dot_general(
            xn, mem96, (((2,), (1,)), ((), ())), preferred_element_type=f32)

        upd = jnp.concatenate(
            [pos_upd, xn.reshape(N_NOISE_ROWS, INPUT_SIZE)], axis=0)
        ss = jnp.sum(upd * upd, axis=1, keepdims=True)
        nrm = jnp.maximum(jnp.sqrt(ss), 1e-12)
        head_ref[...] = upd / nrm


_NC, _NS = 2, 16
_NW = _NC * _NS
MEM_ROWS = OUTPUT_SIZE - NOISE_END
CH = 448
N_CHUNKS = MEM_ROWS // CH
TAIL = MEM_ROWS - N_CHUNKS * CH


def _normalize_rows(buf, base, nrows):
    @plsc.parallel_loop(0, nrows, 1, unroll=4)
    def _row(r):
        rr = base + r
        vecs = [buf[rr, pl.ds(c * 16, 16)] for c in range(INPUT_SIZE // 16)]
        sq = [v * v for v in vecs]
        s0 = (sq[0] + sq[1]) + (sq[2] + sq[3])
        s1 = (sq[4] + sq[5]) + (sq[6] + sq[7])
        ss16 = s0 + s1
        ss = jnp.sum(ss16)
        ssv = jnp.maximum(jax.lax.broadcast_in_dim(ss, (16,), ()), 1e-24)
        i32 = plsc.bitcast(ssv, jnp.int32)
        y = plsc.bitcast(jnp.int32(0x5F3759DF) - (i32 >> 1), jnp.float32)
        y = y * (1.5 - 0.5 * ssv * y * y)
        y = y * (1.5 - 0.5 * ssv * y * y)
        for c in range(INPUT_SIZE // 16):
            buf[rr, pl.ds(c * 16, 16)] = vecs[c] * y


def _sc_body(mem_hbm, out_hbm, buf, in_sems, out_sems):
    wid = lax.axis_index("s") * _NC + lax.axis_index("c")

    n_my = jnp.int32(N_CHUNKS // _NW) + (wid < (N_CHUNKS % _NW)).astype(jnp.int32)

    def off(t):
        return NOISE_END + (wid + t * _NW) * CH

    pltpu.async_copy(mem_hbm.at[pl.ds(off(0), CH)], buf.at[pl.ds(0, CH)],
                     in_sems.at[0])

    def chunk_body(t, carry):
        s = jnp.bitwise_and(t, 1)
        so = 1 - s
        pltpu.make_async_copy(mem_hbm.at[pl.ds(off(t), CH)],
                              buf.at[pl.ds(s * CH, CH)], in_sems.at[s]).wait()

        @pl.when(t >= 1)
        def _():
            pltpu.make_async_copy(buf.at[pl.ds(so * CH, CH)],
                                  out_hbm.at[pl.ds(off(t - 1), CH)],
                                  out_sems.at[so]).wait()

        @pl.when(t + 1 < n_my)
        def _():
            pltpu.async_copy(mem_hbm.at[pl.ds(off(t + 1), CH)],
                             buf.at[pl.ds(so * CH, CH)], in_sems.at[so])

        _normalize_rows(buf, s * CH, CH)
        pltpu.async_copy(buf.at[pl.ds(s * CH, CH)],
                         out_hbm.at[pl.ds(off(t), CH)], out_sems.at[s])
        return carry

    lax.fori_loop(0, n_my, chunk_body, jnp.int32(0))

    sl = jnp.bitwise_and(n_my - 1, 1)
    pltpu.make_async_copy(buf.at[pl.ds(sl * CH, CH)],
                          out_hbm.at[pl.ds(off(n_my - 1), CH)],
                          out_sems.at[sl]).wait()

    @pl.when(wid == _NW - 1)
    def _():
        toff = NOISE_END + N_CHUNKS * CH
        pltpu.sync_copy(mem_hbm.at[pl.ds(toff, TAIL)], buf.at[pl.ds(0, TAIL)])
        _normalize_rows(buf, 0, TAIL)
        pltpu.sync_copy(buf.at[pl.ds(0, TAIL)], out_hbm.at[pl.ds(toff, TAIL)])


_sc_update = functools.partial(
    pl.kernel,
    out_type=jax.ShapeDtypeStruct((OUTPUT_SIZE, INPUT_SIZE), jnp.float32),
    mesh=plsc.VectorSubcoreMesh(core_axis_name="c", subcore_axis_name="s",
                                num_cores=_NC, num_subcores=_NS),
    scratch_types=[pltpu.VMEM((2 * CH, INPUT_SIZE), jnp.float32),
                   pltpu.SemaphoreType.DMA((2,)),
                   pltpu.SemaphoreType.DMA((2,))],
    compiler_params=pltpu.CompilerParams(needs_layout_passes=False),
)(_sc_body)


def kernel(x, y, visible, img_label, memory):
    lab = img_label.astype(jnp.int32).reshape(B, 1)

    grid = ((OUTPUT_SIZE + BM - 1) // BM,)
    similarity, noise_similarity, lwo, head = pl.pallas_call(
        _tc_body,
        grid=grid,
        in_specs=[
            pl.BlockSpec((B, SFD + NUM_NOISE, INPUT_SIZE), lambda i: (0, 0, 0)),
            pl.BlockSpec((B, SFD), lambda i: (0, 0)),
            pl.BlockSpec((B, 1), lambda i: (0, 0)),
            pl.BlockSpec((BM, INPUT_SIZE), lambda i: (i, 0)),
        ],
        out_specs=[
            pl.BlockSpec((B, SFD, BM), lambda i: (0, 0, i)),
            pl.BlockSpec((B, NUM_NOISE, NUM_POS), lambda i: (0, 0, 0)),
            pl.BlockSpec((B, N_CLASSES), lambda i: (0, 0)),
            pl.BlockSpec((NOISE_END, INPUT_SIZE), lambda i: (0, 0)),
        ],
        out_shape=[
            jax.ShapeDtypeStruct((B, SFD, OUTPUT_SIZE), jnp.float32),
            jax.ShapeDtypeStruct((B, NUM_NOISE, NUM_POS), jnp.float32),
            jax.ShapeDtypeStruct((B, N_CLASSES), jnp.float32),
            jax.ShapeDtypeStruct((NOISE_END, INPUT_SIZE), jnp.float32),
        ],
    )(x, visible, lab, memory)

    bank = _sc_update(memory)
    new_memory = jax.lax.dynamic_update_slice(bank, head, (0, 0))

    y_idx = y.astype(jnp.int32)
    return (similarity, y_idx, noise_similarity, lwo, new_memory)


# speedup vs baseline: 1.3978x; 1.0139x over previous
import functools

import jax
import jax.numpy as jnp
from jax import lax
from jax.experimental import pallas as pl
from jax.experimental.pallas import tpu as pltpu
from jax.experimental.pallas import tpu_sc as plsc

INPUT_SIZE = 128
OUTPUT_SIZE = 100000
NUM_POS = 96
NUM_NOISE = 64
SFD = 8
N_CLASSES = 12
MOMENTUM = 0.5
B = 64

BM = 8192
N_NOISE_ROWS = NUM_NOISE * B
NOISE_END = NUM_POS + N_NOISE_ROWS


def _tc_body(x_ref, vis_ref, lab_ref, mem_ref,
             sim_ref, nsim_ref, lwo_ref, head_ref):
    i = pl.program_id(0)
    f32 = jnp.float32

    xpos = x_ref[:, 0:SFD, :]
    mem = mem_ref[...]

    sim_ref[...] = jax.lax.dot_general(
        xpos.astype(jnp.bfloat16), mem.astype(jnp.bfloat16),
        (((2,), (1,)), ((), ())), preferred_element_type=f32)

    @pl.when(i == 0)
    def _():
        lab = lab_ref[...]
        cls = jax.lax.broadcasted_iota(jnp.int32, (B, N_CLASSES), 1)
        eq = (lab == cls).astype(f32)
        cnt = jnp.sum(eq, axis=0, keepdims=True)
        denom = jnp.where(cnt == 0.0, 1.0, cnt)
        lwo = eq / denom
        lwo_ref[...] = lwo

        r0 = jax.lax.broadcasted_iota(jnp.int32, (NUM_POS, N_CLASSES), 0)
        r1 = jax.lax.broadcasted_iota(jnp.int32, (NUM_POS, N_CLASSES), 1)
        rrow = ((r0 // SFD) == r1).astype(f32)
        p1 = jax.lax.dot_general(rrow, lwo, (((1,), (1,)), ((), ())),
                                 preferred_element_type=f32)
        c0 = jax.lax.broadcasted_iota(jnp.int32, (B * SFD, B), 0)
        c1 = jax.lax.broadcasted_iota(jnp.int32, (B * SFD, B), 1)
        rcol = ((c0 // SFD) == c1).astype(f32)
        p2 = jax.lax.dot_general(p1, rcol, (((1,), (1,)), ((), ())),
                                 preferred_element_type=f32)
        m0 = jax.lax.broadcasted_iota(jnp.int32, (NUM_POS, B * SFD), 0)
        m1 = jax.lax.broadcasted_iota(jnp.int32, (NUM_POS, B * SFD), 1)
        pmat = p2 * ((m0 % SFD) == (m1 % SFD)).astype(f32)
        present = jnp.sum(pmat, axis=1, keepdims=True) > 0.5

        xv = (xpos * vis_ref[...][:, :, None]).reshape(B * SFD, INPUT_SIZE)
        get96 = jax.lax.dot_general(pmat, xv, (((1,), (0,)), ((), ())),
                                    preferred_element_type=f32)
        mem96 = mem[0:NUM_POS, :]
        pos_upd = MOMENTUM * mem96 + (1.0 - MOMENTUM) * jnp.where(
            present, get96, mem96)

        xn = x_ref[:, SFD:, :]
        nsim_ref[...] = jax.lax.dot_general(
            xn, mem96, (((2,), (1,)), ((), ())), preferred_element_type=f32)

        upd = jnp.concatenate(
            [pos_upd, xn.reshape(N_NOISE_ROWS, INPUT_SIZE)], axis=0)
        ss = jnp.sum(upd * upd, axis=1, keepdims=True)
        nrm = jnp.maximum(jnp.sqrt(ss), 1e-12)
        head_ref[...] = upd / nrm


_NC, _NS = 2, 16
_NW = _NC * _NS
MEM_ROWS = OUTPUT_SIZE - NOISE_END
CH = 448
N_CHUNKS = MEM_ROWS // CH
TAIL = MEM_ROWS - N_CHUNKS * CH


def _normalize_rows(buf, base, nrows):
    @plsc.parallel_loop(0, nrows, 1, unroll=4)
    def _row(r):
        rr = base + r
        vecs = [buf[rr, pl.ds(c * 16, 16)] for c in range(INPUT_SIZE // 16)]
        sq = [v * v for v in vecs]
        s0 = (sq[0] + sq[1]) + (sq[2] + sq[3])
        s1 = (sq[4] + sq[5]) + (sq[6] + sq[7])
        ss16 = s0 + s1
        ss = jnp.sum(ss16)
        ssv = jnp.maximum(jax.lax.broadcast_in_dim(ss, (16,), ()), 1e-24)
        i32 = plsc.bitcast(ssv, jnp.int32)
        y = plsc.bitcast(jnp.int32(0x5F3759DF) - (i32 >> 1), jnp.float32)
        y = y * (1.5 - 0.5 * ssv * y * y)
        y = y * (1.5 - 0.5 * ssv * y * y)
        for c in range(INPUT_SIZE // 16):
            buf[rr, pl.ds(c * 16, 16)] = vecs[c] * y


def _sc_body(mem_hbm, out_hbm, buf, in_sems, out_sems):
    wid = lax.axis_index("s") * _NC + lax.axis_index("c")

    n_my = jnp.int32(N_CHUNKS // _NW) + (wid < (N_CHUNKS % _NW)).astype(jnp.int32)

    def off(t):
        return NOISE_END + (wid + t * _NW) * CH

    pltpu.async_copy(mem_hbm.at[pl.ds(off(0), CH)], buf.at[pl.ds(0, CH)],
                     in_sems.at[0])

    def chunk_body(t, carry):
        s = jnp.bitwise_and(t, 1)
        so = 1 - s
        pltpu.make_async_copy(mem_hbm.at[pl.ds(off(t), CH)],
                              buf.at[pl.ds(s * CH, CH)], in_sems.at[s]).wait()

        @pl.when(t >= 1)
        def _():
            pltpu.make_async_copy(buf.at[pl.ds(so * CH, CH)],
                                  out_hbm.at[pl.ds(off(t - 1), CH)],
                                  out_sems.at[so]).wait()

        @pl.when(t + 1 < n_my)
        def _():
            pltpu.async_copy(mem_hbm.at[pl.ds(off(t + 1), CH)],
                             buf.at[pl.ds(so * CH, CH)], in_sems.at[so])

        _normalize_rows(buf, s * CH, CH)
        pltpu.async_copy(buf.at[pl.ds(s * CH, CH)],
                         out_hbm.at[pl.ds(off(t), CH)], out_sems.at[s])
        return carry

    lax.fori_loop(0, n_my, chunk_body, jnp.int32(0))

    sl = jnp.bitwise_and(n_my - 1, 1)
    pltpu.make_async_copy(buf.at[pl.ds(sl * CH, CH)],
                          out_hbm.at[pl.ds(off(n_my - 1), CH)],
                          out_sems.at[sl]).wait()

    @pl.when(wid == _NW - 1)
    def _():
        toff = NOISE_END + N_CHUNKS * CH
        pltpu.sync_copy(mem_hbm.at[pl.ds(toff, TAIL)], buf.at[pl.ds(0, TAIL)])
        _normalize_rows(buf, 0, TAIL)
        pltpu.sync_copy(buf.at[pl.ds(0, TAIL)], out_hbm.at[pl.ds(toff, TAIL)])


_sc_update = functools.partial(
    pl.kernel,
    out_type=jax.ShapeDtypeStruct((OUTPUT_SIZE, INPUT_SIZE), jnp.float32),
    mesh=plsc.VectorSubcoreMesh(core_axis_name="c", subcore_axis_name="s",
                                num_cores=_NC, num_subcores=_NS),
    scratch_types=[pltpu.VMEM((2 * CH, INPUT_SIZE), jnp.float32),
                   pltpu.SemaphoreType.DMA((2,)),
                   pltpu.SemaphoreType.DMA((2,))],
    compiler_params=pltpu.CompilerParams(needs_layout_passes=False),
)(_sc_body)


def kernel(x, y, visible, img_label, memory):
    lab = img_label.astype(jnp.int32).reshape(B, 1)

    grid = ((OUTPUT_SIZE + BM - 1) // BM,)
    similarity, noise_similarity, lwo, head = pl.pallas_call(
        _tc_body,
        grid=grid,
        in_specs=[
            pl.BlockSpec((B, SFD + NUM_NOISE, INPUT_SIZE), lambda i: (0, 0, 0)),
            pl.BlockSpec((B, SFD), lambda i: (0, 0)),
            pl.BlockSpec((B, 1), lambda i: (0, 0)),
            pl.BlockSpec((BM, INPUT_SIZE), lambda i: (i, 0)),
        ],
        out_specs=[
            pl.BlockSpec((B, SFD, BM), lambda i: (0, 0, i)),
            pl.BlockSpec((B, NUM_NOISE, NUM_POS), lambda i: (0, 0, 0)),
            pl.BlockSpec((B, N_CLASSES), lambda i: (0, 0)),
            pl.BlockSpec((NOISE_END, INPUT_SIZE), lambda i: (0, 0)),
        ],
        out_shape=[
            jax.ShapeDtypeStruct((B, SFD, OUTPUT_SIZE), jnp.float32),
            jax.ShapeDtypeStruct((B, NUM_NOISE, NUM_POS), jnp.float32),
            jax.ShapeDtypeStruct((B, N_CLASSES), jnp.float32),
            jax.ShapeDtypeStruct((NOISE_END, INPUT_SIZE), jnp.float32),
        ],
    )(x, visible, lab, memory)

    bank = _sc_update(memory)
    new_memory = jax.lax.dynamic_update_slice(bank, head, (0, 0))

    y_idx = y.astype(jnp.int32)
    return (similarity, y_idx, noise_similarity, lwo, new_memory)


# speedup vs baseline: 1.4010x; 1.0023x over previous
import functools

import jax
import jax.numpy as jnp
from jax import lax
from jax.experimental import pallas as pl
from jax.experimental.pallas import tpu as pltpu
from jax.experimental.pallas import tpu_sc as plsc

INPUT_SIZE = 128
OUTPUT_SIZE = 100000
NUM_POS = 96
NUM_NOISE = 64
SFD = 8
N_CLASSES = 12
MOMENTUM = 0.5
B = 64

BM = 8192
N_NOISE_ROWS = NUM_NOISE * B
NOISE_END = NUM_POS + N_NOISE_ROWS


def _tc_body(x_ref, vis_ref, lab_ref, mem_ref,
             sim_ref, nsim_ref, lwo_ref, head_ref):
    i = pl.program_id(0)
    f32 = jnp.float32

    xpos = x_ref[:, 0:SFD, :]
    mem = mem_ref[...]

    sim_ref[...] = jax.lax.dot_general(
        xpos.astype(jnp.bfloat16), mem.astype(jnp.bfloat16),
        (((2,), (1,)), ((), ())), preferred_element_type=f32)

    @pl.when(i == 0)
    def _():
        lab = lab_ref[...]
        cls = jax.lax.broadcasted_iota(jnp.int32, (B, N_CLASSES), 1)
        eq = (lab == cls).astype(f32)
        cnt = jnp.sum(eq, axis=0, keepdims=True)
        denom = jnp.where(cnt == 0.0, 1.0, cnt)
        lwo = eq / denom
        lwo_ref[...] = lwo

        r0 = jax.lax.broadcasted_iota(jnp.int32, (NUM_POS, N_CLASSES), 0)
        r1 = jax.lax.broadcasted_iota(jnp.int32, (NUM_POS, N_CLASSES), 1)
        rrow = ((r0 // SFD) == r1).astype(f32)
        p1 = jax.lax.dot_general(rrow, lwo, (((1,), (1,)), ((), ())),
                                 preferred_element_type=f32)
        c0 = jax.lax.broadcasted_iota(jnp.int32, (B * SFD, B), 0)
        c1 = jax.lax.broadcasted_iota(jnp.int32, (B * SFD, B), 1)
        rcol = ((c0 // SFD) == c1).astype(f32)
        p2 = jax.lax.dot_general(p1, rcol, (((1,), (1,)), ((), ())),
                                 preferred_element_type=f32)
        m0 = jax.lax.broadcasted_iota(jnp.int32, (NUM_POS, B * SFD), 0)
        m1 = jax.lax.broadcasted_iota(jnp.int32, (NUM_POS, B * SFD), 1)
        pmat = p2 * ((m0 % SFD) == (m1 % SFD)).astype(f32)
        present = jnp.sum(pmat, axis=1, keepdims=True) > 0.5

        xv = (xpos * vis_ref[...][:, :, None]).reshape(B * SFD, INPUT_SIZE)
        get96 = jax.lax.dot_general(pmat, xv, (((1,), (0,)), ((), ())),
                                    preferred_element_type=f32)
        mem96 = mem[0:NUM_POS, :]
        pos_upd = MOMENTUM * mem96 + (1.0 - MOMENTUM) * jnp.where(
            present, get96, mem96)

        xn = x_ref[:, SFD:, :]
        nsim_ref[...] = jax.lax.dot_general(
            xn, mem96, (((2,), (1,)), ((), ())), preferred_element_type=f32)

        upd = jnp.concatenate(
            [pos_upd, xn.reshape(N_NOISE_ROWS, INPUT_SIZE)], axis=0)
        ss = jnp.sum(upd * upd, axis=1, keepdims=True)
        nrm = jnp.maximum(jnp.sqrt(ss), 1e-12)
        head_ref[...] = upd / nrm


_NC, _NS = 2, 16
_NW = _NC * _NS
MEM_ROWS = OUTPUT_SIZE - NOISE_END
CH = 320
N_CHUNKS = MEM_ROWS // CH
TAIL = MEM_ROWS - N_CHUNKS * CH
NBUF = 3


def _normalize_rows(buf, base, nrows):
    @plsc.parallel_loop(0, nrows, 1, unroll=4)
    def _row(r):
        rr = base + r
        vecs = [buf[rr, pl.ds(c * 16, 16)] for c in range(INPUT_SIZE // 16)]
        sq = [v * v for v in vecs]
        s0 = (sq[0] + sq[1]) + (sq[2] + sq[3])
        s1 = (sq[4] + sq[5]) + (sq[6] + sq[7])
        ss16 = s0 + s1
        ss = jnp.sum(ss16)
        ssv = jnp.maximum(jax.lax.broadcast_in_dim(ss, (16,), ()), 1e-24)
        i32 = plsc.bitcast(ssv, jnp.int32)
        y = plsc.bitcast(jnp.int32(0x5F3759DF) - (i32 >> 1), jnp.float32)
        y = y * (1.5 - 0.5 * ssv * y * y)
        y = y * (1.5 - 0.5 * ssv * y * y)
        for c in range(INPUT_SIZE // 16):
            buf[rr, pl.ds(c * 16, 16)] = vecs[c] * y


def _sc_body(mem_hbm, out_hbm, buf, in_sems, out_sems):
    wid = lax.axis_index("s") * _NC + lax.axis_index("c")

    n_my = jnp.int32(N_CHUNKS // _NW) + (wid < (N_CHUNKS % _NW)).astype(jnp.int32)

    def off(t):
        return NOISE_END + (wid + t * _NW) * CH

    pltpu.async_copy(mem_hbm.at[pl.ds(off(0), CH)], buf.at[pl.ds(0, CH)],
                     in_sems.at[0])

    @pl.when(n_my >= 2)
    def _():
        pltpu.async_copy(mem_hbm.at[pl.ds(off(1), CH)], buf.at[pl.ds(CH, CH)],
                         in_sems.at[1])

    def chunk_body(t, carry):
        s = lax.rem(t, NBUF)
        sn = lax.rem(t + 2, NBUF)
        pltpu.make_async_copy(mem_hbm.at[pl.ds(off(t), CH)],
                              buf.at[pl.ds(s * CH, CH)], in_sems.at[s]).wait()

        @pl.when(jnp.logical_and(t + 2 < n_my, t >= 1))
        def _():
            pltpu.make_async_copy(buf.at[pl.ds(sn * CH, CH)],
                                  out_hbm.at[pl.ds(off(t - 1), CH)],
                                  out_sems.at[sn]).wait()

        @pl.when(t + 2 < n_my)
        def _():
            pltpu.async_copy(mem_hbm.at[pl.ds(off(t + 2), CH)],
                             buf.at[pl.ds(sn * CH, CH)], in_sems.at[sn])

        _normalize_rows(buf, s * CH, CH)
        pltpu.async_copy(buf.at[pl.ds(s * CH, CH)],
                         out_hbm.at[pl.ds(off(t), CH)], out_sems.at[s])
        return carry

    lax.fori_loop(0, n_my, chunk_body, jnp.int32(0))

    def drain(k, carry):
        t = n_my - 3 + k
        sl = lax.rem(t, NBUF)
        pltpu.make_async_copy(buf.at[pl.ds(sl * CH, CH)],
                              out_hbm.at[pl.ds(off(t), CH)],
                              out_sems.at[sl]).wait()
        return carry

    lax.fori_loop(0, 3, drain, jnp.int32(0))

    @pl.when(wid == _NW - 1)
    def _():
        toff = NOISE_END + N_CHUNKS * CH
        pltpu.sync_copy(mem_hbm.at[pl.ds(toff, TAIL)], buf.at[pl.ds(0, TAIL)])
        _normalize_rows(buf, 0, TAIL)
        pltpu.sync_copy(buf.at[pl.ds(0, TAIL)], out_hbm.at[pl.ds(toff, TAIL)])


_sc_update = functools.partial(
    pl.kernel,
    out_type=jax.ShapeDtypeStruct((OUTPUT_SIZE, INPUT_SIZE), jnp.float32),
    mesh=plsc.VectorSubcoreMesh(core_axis_name="c", subcore_axis_name="s",
                                num_cores=_NC, num_subcores=_NS),
    scratch_types=[pltpu.VMEM((NBUF * CH, INPUT_SIZE), jnp.float32),
                   pltpu.SemaphoreType.DMA((NBUF,)),
                   pltpu.SemaphoreType.DMA((NBUF,))],
    compiler_params=pltpu.CompilerParams(needs_layout_passes=False),
)(_sc_body)


def kernel(x, y, visible, img_label, memory):
    lab = img_label.astype(jnp.int32).reshape(B, 1)

    grid = ((OUTPUT_SIZE + BM - 1) // BM,)
    similarity, noise_similarity, lwo, head = pl.pallas_call(
        _tc_body,
        grid=grid,
        in_specs=[
            pl.BlockSpec((B, SFD + NUM_NOISE, INPUT_SIZE), lambda i: (0, 0, 0)),
            pl.BlockSpec((B, SFD), lambda i: (0, 0)),
            pl.BlockSpec((B, 1), lambda i: (0, 0)),
            pl.BlockSpec((BM, INPUT_SIZE), lambda i: (i, 0)),
        ],
        out_specs=[
            pl.BlockSpec((B, SFD, BM), lambda i: (0, 0, i)),
            pl.BlockSpec((B, NUM_NOISE, NUM_POS), lambda i: (0, 0, 0)),
            pl.BlockSpec((B, N_CLASSES), lambda i: (0, 0)),
            pl.BlockSpec((NOISE_END, INPUT_SIZE), lambda i: (0, 0)),
        ],
        out_shape=[
            jax.ShapeDtypeStruct((B, SFD, OUTPUT_SIZE), jnp.float32),
            jax.ShapeDtypeStruct((B, NUM_NOISE, NUM_POS), jnp.float32),
            jax.ShapeDtypeStruct((B, N_CLASSES), jnp.float32),
            jax.ShapeDtypeStruct((NOISE_END, INPUT_SIZE), jnp.float32),
        ],
    )(x, visible, lab, memory)

    bank = _sc_update(memory)
    new_memory = jax.lax.dynamic_update_slice(bank, head, (0, 0))

    y_idx = y.astype(jnp.int32)
    return (similarity, y_idx, noise_similarity, lwo, new_memory)
